# baseline (speedup 1.0000x reference)
import functools

import jax
import jax.numpy as jnp
from jax import lax
from jax.experimental import pallas as pl
from jax.experimental.pallas import tpu as pltpu
from jax.experimental.pallas import tpu_sc as plsc

_CHUNK = 512
_NBUF = 5
_L16 = 16


@functools.lru_cache(maxsize=None)
def _build_flatten(batch: int, seqlen: int):
    info = plsc.get_sparse_core_info()
    nw = info.num_cores * info.num_subcores
    nc = info.num_cores
    assert batch % nw == 0
    s_per_w = batch // nw
    per_w = s_per_w * seqlen
    assert seqlen % 8 == 0

    mesh = plsc.VectorSubcoreMesh(core_axis_name="c", subcore_axis_name="s")

    @functools.partial(
        pl.kernel,
        out_type=jax.ShapeDtypeStruct((batch * seqlen,), jnp.int32),
        mesh=mesh,
        scratch_types=[
            pltpu.VMEM((s_per_w, seqlen), jnp.int32),
            pltpu.VMEM((per_w,), jnp.int32),
        ],
        compiler_params=pltpu.CompilerParams(use_tc_tiling_on_sc=True),
    )
    def flatten_kernel(ids_hbm, out_hbm, raw_v, flat_v):
        wid = lax.axis_index("s") * nc + lax.axis_index("c")
        pltpu.sync_copy(ids_hbm.at[pl.ds(wid * s_per_w, s_per_w)], raw_v)

        starts = list(range(0, seqlen - _L16, _L16)) + [seqlen - _L16]

        @pl.loop(0, s_per_w)
        def _(i):
            base = pl.multiple_of(seqlen * i, 8)
            for c in starts:
                flat_v[pl.ds(base + c, _L16)] = raw_v[i, pl.ds(c, _L16)]

        pltpu.sync_copy(flat_v, out_hbm.at[pl.ds(wid * per_w, per_w)])

    return flatten_kernel


@functools.lru_cache(maxsize=None)
def _build_gather(n_tokens: int, vocab: int, embed: int):
    info = plsc.get_sparse_core_info()
    nw = info.num_cores * info.num_subcores
    assert n_tokens % (nw * _CHUNK) == 0
    per_w = n_tokens // nw
    nchunk = per_w // _CHUNK
    assert nchunk % _NBUF == 0
    nc = info.num_cores

    mesh = plsc.VectorSubcoreMesh(core_axis_name="c", subcore_axis_name="s")

    @functools.partial(
        pl.kernel,
        out_type=jax.ShapeDtypeStruct((n_tokens, embed), jnp.int32),
        mesh=mesh,
        scratch_types=[
            pltpu.VMEM((per_w,), jnp.int32),
            pltpu.VMEM((_NBUF, _CHUNK, embed), jnp.int32),
            pltpu.SemaphoreType.DMA((_NBUF,)),
            pltpu.SemaphoreType.DMA((_NBUF,)),
        ],
        compiler_params=pltpu.CompilerParams(use_tc_tiling_on_sc=False),
    )
    def gather_kernel(ids_hbm, table_hbm, out_hbm, idx_v, rows_v, gsem, ssem):
        wid = lax.axis_index("s") * nc + lax.axis_index("c")
        row0 = wid * per_w

        pltpu.sync_copy(ids_hbm.at[pl.ds(row0, per_w)], idx_v)

        def gather(buf, chunk):
            return pltpu.make_async_copy(
                table_hbm.at[idx_v.at[pl.ds(chunk * _CHUNK, _CHUNK)]],
                rows_v.at[buf], gsem.at[buf])

        def scatter(buf, chunk):
            return pltpu.make_async_copy(
                rows_v.at[buf],
                out_hbm.at[pl.ds(row0 + chunk * _CHUNK, _CHUNK)],
                ssem.at[buf])

        for b in range(_NBUF):
            gather(b, b).start()

        @pl.loop(0, nchunk, step=_NBUF)
        def _(j):
            for b in range(_NBUF):
                gather(b, j + b).wait()
                scatter(b, j + b).start()
            for b in range(_NBUF):
                scatter(b, j + b).wait()

                @pl.when(j + b + _NBUF < nchunk)
                def _():
                    gather(b, j + b + _NBUF).start()

    return gather_kernel


@functools.lru_cache(maxsize=None)
def _build_depad_table(vocab: int, embed: int):
    info = plsc.get_sparse_core_info()
    nw = info.num_cores * info.num_subcores
    nc = info.num_cores
    blk = 320
    assert vocab % blk == 0
    nblk = vocab // blk
    nbuf = 2

    mesh = plsc.VectorSubcoreMesh(core_axis_name="c", subcore_axis_name="s")

    @functools.partial(
        pl.kernel,
        out_type=jax.ShapeDtypeStruct((vocab * embed,), jnp.int32),
        mesh=mesh,
        scratch_types=[
            pltpu.VMEM((nbuf, blk, embed), jnp.int32),
            pltpu.VMEM((nbuf, blk * embed), jnp.int32),
            pltpu.SemaphoreType.DMA((nbuf,)),
            pltpu.SemaphoreType.DMA((nbuf,)),
        ],
        compiler_params=pltpu.CompilerParams(use_tc_tiling_on_sc=True),
    )
    def depad_kernel(table_hbm, out_hbm, raw_v, flat_v, isem, osem):
        wid = lax.axis_index("s") * nc + lax.axis_index("c")
        nm = (nblk - 1 - wid) // nw + 1

        def load(buf, m):
            k = (wid + m * nw) * blk
            return pltpu.make_async_copy(
                table_hbm.at[pl.ds(k, blk)], raw_v.at[buf], isem.at[buf])

        def store(buf, m):
            k = (wid + m * nw) * (blk * embed)
            return pltpu.make_async_copy(
                flat_v.at[buf],
                out_hbm.at[pl.ds(k, blk * embed)],
                osem.at[buf])

        load(0, 0).start()

        @pl.when(nm > 1)
        def _():
            load(1, 1).start()

        @pl.loop(0, nblk // nw + 1, step=nbuf)
        def _(mm):
            for b in range(nbuf):
                m = mm + b

                @pl.when(m < nm)
                def _(b=b, m=m):
                    load(b, m).wait()

                    @pl.when(m >= nbuf)
                    def _():
                        store(b, m - nbuf).wait()

                    @pl.loop(0, blk)
                    def _(t, _b=b):
                        dst = pl.multiple_of(embed * t, 8)
                        for h in range(0, embed, _L16):
                            flat_v[_b, pl.ds(dst + h, _L16)] = (
                                raw_v[_b, t, pl.ds(h, _L16)])

                    store(b, m).start()

                    @pl.when(m + nbuf < nm)
                    def _():
                        load(b, m + nbuf).start()

        for b in range(nbuf):
            last = nm - 1 - ((nm - 1 - b) % nbuf)

            @pl.when(last >= 0)
            def _(b=b, last=last):
                store(b, last).wait()

    return depad_kernel


@functools.lru_cache(maxsize=None)
def _build_repack(batch: int, seqlen: int, embed: int):
    info = plsc.get_sparse_core_info()
    nw = info.num_cores * info.num_subcores
    nc = info.num_cores
    assert batch % nw == 0
    s_per_w = batch // nw
    nbuf = 2
    words_per_seq = seqlen * embed

    mesh = plsc.VectorSubcoreMesh(core_axis_name="c", subcore_axis_name="s")

    @functools.partial(
        pl.kernel,
        out_type=jax.ShapeDtypeStruct((batch * seqlen, embed), jnp.int32),
        mesh=mesh,
        scratch_types=[
            pltpu.VMEM((nbuf, words_per_seq), jnp.int32),
            pltpu.VMEM((nbuf, seqlen, embed), jnp.int32),
            pltpu.SemaphoreType.DMA((nbuf,)),
            pltpu.SemaphoreType.DMA((nbuf,)),
        ],
        compiler_params=pltpu.CompilerParams(use_tc_tiling_on_sc=True),
    )
    def repack_kernel(flat_hbm, out_hbm, in_v, line_v, isem, osem):
        wid = lax.axis_index("s") * nc + lax.axis_index("c")
        seq0 = wid * s_per_w

        def load(buf, j):
            return pltpu.make_async_copy(
                flat_hbm.at[pl.ds((seq0 + j) * words_per_seq, words_per_seq)],
                in_v.at[buf], isem.at[buf])

        def store(buf, j):
            return pltpu.make_async_copy(
                line_v.at[buf],
                out_hbm.at[pl.ds((seq0 + j) * seqlen, seqlen)],
                osem.at[buf])

        for b in range(nbuf):
            load(b, b).start()

        @pl.loop(0, s_per_w, step=nbuf)
        def _(j):
            for b in range(nbuf):
                load(b, j + b).wait()

                @pl.when(j + b >= nbuf)
                def _():
                    store(b, j + b - nbuf).wait()

                @pl.loop(0, seqlen)
                def _(t, _b=b):
                    src = pl.multiple_of(embed * t, 8)
                    for h in range(0, embed, _L16):
                        line_v[_b, t, pl.ds(h, _L16)] = (
                            in_v[_b, pl.ds(src + h, _L16)])

                store(b, j + b).start()

                @pl.when(j + b + nbuf < s_per_w)
                def _():
                    load(b, j + b + nbuf).start()

        for b in range(nbuf):
            store(b, s_per_w - nbuf + b).wait()

    return repack_kernel


def kernel(input_ids, embedding_table):
    b, l = input_ids.shape
    vocab, embed = embedding_table.shape
    ids_flat = _build_flatten(b, l)(input_ids)
    table_i = jax.lax.bitcast_convert_type(embedding_table, jnp.int32)
    table_c = _build_depad_table(vocab, embed)(table_i)
    out = _build_gather(b * l, vocab, embed)(
        ids_flat, table_c.reshape(vocab, embed))
    out = _build_repack(b, l, embed)(out.reshape(b * l * embed))
    return jax.lax.bitcast_convert_type(out, jnp.float32).reshape(b, l, embed)


# speedup vs baseline: 1.4343x; 1.4343x over previous
import functools

import jax
import jax.numpy as jnp
from jax import lax
from jax.experimental import pallas as pl
from jax.experimental.pallas import tpu as pltpu
from jax.experimental.pallas import tpu_sc as plsc

_CHUNK = 512
_NBUF = 5
_L16 = 16


@functools.lru_cache(maxsize=None)
def _build_flatten(batch: int, seqlen: int):
    info = plsc.get_sparse_core_info()
    nw = info.num_cores * info.num_subcores
    nc = info.num_cores
    assert batch % nw == 0
    s_per_w = batch // nw
    per_w = s_per_w * seqlen
    assert seqlen % 8 == 0

    mesh = plsc.VectorSubcoreMesh(core_axis_name="c", subcore_axis_name="s")

    @functools.partial(
        pl.kernel,
        out_type=jax.ShapeDtypeStruct((batch * seqlen,), jnp.int32),
        mesh=mesh,
        scratch_types=[
            pltpu.VMEM((s_per_w, seqlen), jnp.int32),
            pltpu.VMEM((per_w,), jnp.int32),
        ],
        compiler_params=pltpu.CompilerParams(use_tc_tiling_on_sc=True),
    )
    def flatten_kernel(ids_hbm, out_hbm, raw_v, flat_v):
        wid = lax.axis_index("s") * nc + lax.axis_index("c")
        pltpu.sync_copy(ids_hbm.at[pl.ds(wid * s_per_w, s_per_w)], raw_v)

        starts = list(range(0, seqlen - _L16, _L16)) + [seqlen - _L16]

        @pl.loop(0, s_per_w)
        def _(i):
            base = pl.multiple_of(seqlen * i, 8)
            for c in starts:
                flat_v[pl.ds(base + c, _L16)] = raw_v[i, pl.ds(c, _L16)]

        pltpu.sync_copy(flat_v, out_hbm.at[pl.ds(wid * per_w, per_w)])

    return flatten_kernel


@functools.lru_cache(maxsize=None)
def _build_gather(n_tokens: int, vocab: int, embed: int):
    info = plsc.get_sparse_core_info()
    nw = info.num_cores * info.num_subcores
    assert n_tokens % (nw * _CHUNK) == 0
    per_w = n_tokens // nw
    nchunk = per_w // _CHUNK
    assert nchunk % _NBUF == 0
    nc = info.num_cores

    mesh = plsc.VectorSubcoreMesh(core_axis_name="c", subcore_axis_name="s")

    @functools.partial(
        pl.kernel,
        out_type=jax.ShapeDtypeStruct((n_tokens, embed), jnp.float32),
        mesh=mesh,
        scratch_types=[
            pltpu.VMEM((per_w,), jnp.int32),
            pltpu.VMEM((_NBUF, _CHUNK, embed), jnp.float32),
            pltpu.SemaphoreType.DMA((_NBUF,)),
            pltpu.SemaphoreType.DMA((_NBUF,)),
        ],
        compiler_params=pltpu.CompilerParams(use_tc_tiling_on_sc=False),
    )
    def gather_kernel(ids_hbm, table_hbm, out_hbm, idx_v, rows_v, gsem, ssem):
        wid = lax.axis_index("s") * nc + lax.axis_index("c")
        row0 = wid * per_w

        pltpu.sync_copy(ids_hbm.at[pl.ds(row0, per_w)], idx_v)

        def gather(buf, chunk):
            return pltpu.make_async_copy(
                table_hbm.at[idx_v.at[pl.ds(chunk * _CHUNK, _CHUNK)]],
                rows_v.at[buf], gsem.at[buf])

        def scatter(buf, chunk):
            return pltpu.make_async_copy(
                rows_v.at[buf],
                out_hbm.at[pl.ds(row0 + chunk * _CHUNK, _CHUNK)],
                ssem.at[buf])

        for b in range(_NBUF):
            gather(b, b).start()

        @pl.loop(0, nchunk, step=_NBUF)
        def _(j):
            for b in range(_NBUF):
                gather(b, j + b).wait()
                scatter(b, j + b).start()
            for b in range(_NBUF):
                scatter(b, j + b).wait()

                @pl.when(j + b + _NBUF < nchunk)
                def _():
                    gather(b, j + b + _NBUF).start()

    return gather_kernel


@functools.lru_cache(maxsize=None)
def _build_repack(batch: int, seqlen: int, embed: int):
    info = plsc.get_sparse_core_info()
    nw = info.num_cores * info.num_subcores
    nc = info.num_cores
    assert batch % nw == 0
    s_per_w = batch // nw
    nbuf = 2
    words_per_seq = seqlen * embed

    mesh = plsc.VectorSubcoreMesh(core_axis_name="c", subcore_axis_name="s")

    @functools.partial(
        pl.kernel,
        out_type=jax.ShapeDtypeStruct((batch * seqlen, embed), jnp.float32),
        mesh=mesh,
        scratch_types=[
            pltpu.VMEM((nbuf, words_per_seq), jnp.float32),
            pltpu.VMEM((nbuf, seqlen, embed), jnp.float32),
            pltpu.SemaphoreType.DMA((nbuf,)),
            pltpu.SemaphoreType.DMA((nbuf,)),
        ],
        compiler_params=pltpu.CompilerParams(use_tc_tiling_on_sc=True),
    )
    def repack_kernel(flat_hbm, out_hbm, in_v, line_v, isem, osem):
        wid = lax.axis_index("s") * nc + lax.axis_index("c")
        seq0 = wid * s_per_w

        def load(buf, j):
            return pltpu.make_async_copy(
                flat_hbm.at[pl.ds((seq0 + j) * words_per_seq, words_per_seq)],
                in_v.at[buf], isem.at[buf])

        def store(buf, j):
            return pltpu.make_async_copy(
                line_v.at[buf],
                out_hbm.at[pl.ds((seq0 + j) * seqlen, seqlen)],
                osem.at[buf])

        for b in range(nbuf):
            load(b, b).start()

        @pl.loop(0, s_per_w, step=nbuf)
        def _(j):
            for b in range(nbuf):
                load(b, j + b).wait()

                @pl.when(j + b >= nbuf)
                def _():
                    store(b, j + b - nbuf).wait()

                @pl.loop(0, seqlen)
                def _(t, _b=b):
                    src = pl.multiple_of(embed * t, 8)
                    for h in range(0, embed, _L16):
                        line_v[_b, t, pl.ds(h, _L16)] = (
                            in_v[_b, pl.ds(src + h, _L16)])

                store(b, j + b).start()

                @pl.when(j + b + nbuf < s_per_w)
                def _():
                    load(b, j + b + nbuf).start()

        for b in range(nbuf):
            store(b, s_per_w - nbuf + b).wait()

    return repack_kernel


def kernel(input_ids, embedding_table):
    b, l = input_ids.shape
    vocab, embed = embedding_table.shape
    ids_flat = _build_flatten(b, l)(input_ids)
    out = _build_gather(b * l, vocab, embed)(ids_flat, embedding_table)
    out = _build_repack(b, l, embed)(out.reshape(b * l * embed))
    return out.reshape(b, l, embed)
